# Initial kernel scaffold; baseline (speedup 1.0000x reference)
import jax
import jax.numpy as jnp
from jax.experimental import pallas as pl


def kernel(x, conv1_w, conv1_b, conv2_w, conv2_b, fc1_w, fc1_b, fc2_w, fc2_b):
    raise NotImplementedError("write your pallas kernel here")


# speedup vs baseline: 4.4679x; 4.4679x over previous
import functools

import jax
import jax.numpy as jnp
from jax.experimental import pallas as pl
from jax.experimental.pallas import tpu as pltpu

_F = 12
_TB = 128
_HALVES = 2


def _round_up(a, m):
    return (a + m - 1) // m * m


def _fused_kernel(x_ref, w1_ref, b1_ref, w2_ref, b2_ref,
                  f1_ref, f1b_ref, f2_ref, f2b_ref, o_ref, *, n_ch):
    x2 = x_ref[0]
    rows, lanes = x2.shape
    zcol = jnp.zeros((rows, _TB), jnp.float32)

    xm = jnp.concatenate([zcol, x2[:, :-_TB]], axis=1)
    xp = jnp.concatenate([x2[:, _TB:], zcol], axis=1)

    a0 = a1 = a2 = None
    for c in range(n_ch):
        h1 = jnp.maximum(
            w1_ref[3 * c] * xm + w1_ref[3 * c + 1] * x2
            + w1_ref[3 * c + 2] * xp + b1_ref[c], 0.0)
        if c == 0:
            a0 = w2_ref[0] * h1
            a1 = w2_ref[1] * h1
            a2 = w2_ref[2] * h1
        else:
            a0 = a0 + w2_ref[3 * c] * h1
            a1 = a1 + w2_ref[3 * c + 1] * h1
            a2 = a2 + w2_ref[3 * c + 2] * h1

    y2 = jnp.maximum(
        jnp.concatenate([zcol, a0[:, :-_TB]], axis=1) + a1
        + jnp.concatenate([a2[:, _TB:], zcol], axis=1) + b2_ref[0], 0.0)

    h = jnp.maximum(
        jnp.dot(f1_ref[...], y2, preferred_element_type=jnp.float32)
        + f1b_ref[...], 0.0)
    o = (jnp.dot(f2_ref[...], h, preferred_element_type=jnp.float32)
         + f2b_ref[...])

    out0 = o[0:2]
    out1 = o[2:4]
    mid = 0.5 * (out0 + jnp.concatenate([out0[:, :_TB], out1[:, :-_TB]], axis=1))
    o_ref[0] = jnp.concatenate([mid, out1[:, -_TB:]], axis=1)


def kernel(x, conv1_w, conv1_b, conv2_w, conv2_b, fc1_w, fc1_b, fc2_w, fc2_b):
    N, C, S, F = x.shape
    assert C == 1 and F == _F
    n_ch = conv1_w.shape[0]
    hid = fc1_w.shape[0]

    nb = _HALVES * _TB
    npad = _round_up(max(N, 1), nb)
    nblocks = npad // nb

    xs = x[:, 0, :, :].astype(jnp.float32)
    if npad != N:
        xs = jnp.pad(xs, ((0, npad - N), (0, 0), (0, 0)))
    xt = (xs.reshape(nblocks, _HALVES, _TB, S, F)
          .transpose(0, 1, 4, 3, 2)
          .reshape(nblocks, _HALVES * F, S * _TB))

    w1_k = conv1_w.reshape(-1).astype(jnp.float32)
    b1_k = conv1_b.reshape(-1).astype(jnp.float32)
    w2_k = conv2_w.reshape(-1).astype(jnp.float32)
    b2_k = conv2_b.reshape(-1).astype(jnp.float32)

    f1w = fc1_w.astype(jnp.float32)
    f1 = jnp.zeros((2 * hid, 2 * F), jnp.float32)
    f1 = f1.at[:hid, :F].set(f1w).at[hid:, F:].set(f1w)
    f1b = jnp.concatenate([fc1_b, fc1_b]).astype(jnp.float32).reshape(2 * hid, 1)
    f2w = fc2_w.astype(jnp.float32)
    f2 = jnp.zeros((4, 2 * hid), jnp.float32)
    f2 = (f2.at[0, :hid].set(f2w[0]).at[1, hid:].set(f2w[0])
          .at[2, :hid].set(f2w[1]).at[3, hid:].set(f2w[1]))
    f2b = jnp.stack([fc2_b[0], fc2_b[0], fc2_b[1], fc2_b[1]]).reshape(4, 1)

    smem = pl.BlockSpec(memory_space=pltpu.MemorySpace.SMEM)

    out = pl.pallas_call(
        functools.partial(_fused_kernel, n_ch=n_ch),
        out_shape=jax.ShapeDtypeStruct((nblocks, _HALVES, (S + 1) * _TB),
                                       jnp.float32),
        grid=(nblocks,),
        in_specs=[
            pl.BlockSpec((1, _HALVES * F, S * _TB), lambda b: (b, 0, 0)),
            smem, smem, smem, smem,
            pl.BlockSpec((2 * hid, 2 * F), lambda b: (0, 0)),
            pl.BlockSpec((2 * hid, 1), lambda b: (0, 0)),
            pl.BlockSpec((4, 2 * hid), lambda b: (0, 0)),
            pl.BlockSpec((4, 1), lambda b: (0, 0)),
        ],
        out_specs=pl.BlockSpec((1, _HALVES, (S + 1) * _TB),
                               lambda b: (b, 0, 0)),
        compiler_params=pltpu.CompilerParams(
            dimension_semantics=("parallel",),
            vmem_limit_bytes=64 * 1024 * 1024),
    )(xt, w1_k, b1_k, w2_k, b2_k, f1, f1b, f2, f2b)

    res = (out.reshape(nblocks, _HALVES, S + 1, _TB)
           .transpose(0, 1, 3, 2)
           .reshape(npad, S + 1))
    return res[:N]
